# baseline (speedup 1.0000x reference)
import functools

import jax
import jax.numpy as jnp
from jax import lax
from jax.experimental import pallas as pl
from jax.experimental.pallas import tpu as pltpu
from jax.experimental.pallas import tpu_sc as plsc

_NW = 32
_SC_MESH = dict(core_axis_name="c", subcore_axis_name="s")


def _sc_chunk(bpw, words):
    cb = bpw
    while cb * words * 4 > 320 * 1024:
        cb //= 2
    return cb


def _sc_gather(parts, idx):
    table = _pad128(parts[0] if len(parts) == 1 else jnp.concatenate(parts, axis=1))
    B = idx.shape[0]
    D = table.shape[1]
    bpw = B // _NW
    cb = _sc_chunk(bpw, D)
    nch = bpw // cb
    mesh = plsc.VectorSubcoreMesh(**_SC_MESH)

    @functools.partial(
        pl.kernel, mesh=mesh,
        out_type=jax.ShapeDtypeStruct((B, D), jnp.float32),
        scratch_types=[
            pltpu.VMEM((bpw,), jnp.int32),
            pltpu.VMEM((cb, D), jnp.float32),
            pltpu.SemaphoreType.DMA,
        ],
    )
    def gk(tab, idx_hbm, out, idx_v, rows, sem):
        wid = lax.axis_index("s") * 2 + lax.axis_index("c")
        base = wid * bpw
        pltpu.sync_copy(idx_hbm.at[pl.ds(base, bpw)], idx_v)
        for c in range(nch):
            pltpu.async_copy(tab.at[idx_v.at[pl.ds(c * cb, cb)]], rows, sem).wait()
            pltpu.sync_copy(rows, out.at[pl.ds(base + c * cb, cb)])

    return gk(table, idx)


_QT = 128
_CH = 128
_KPAD = 32


def _knn_body(k, NCH, CH, q_ref, rt_ref, oi_ref, os_ref, keys_ref):
    q = q_ref[...]
    q2 = jnp.sum(q * q, axis=1, keepdims=True)
    QT = q.shape[0]
    IMAX = jnp.int32(0x7FFFFFFF)

    def dist_chunk(j2, carry):
        for u in range(4):
            j = 4 * j2 + u
            rt = rt_ref[j]
            r2 = jnp.sum(rt * rt, axis=0, keepdims=True)
            dot = jnp.dot(q, rt, preferred_element_type=jnp.float32)
            d = jnp.maximum((q2 + r2) - 2.0 * dot, 0.0)
            keys_ref[j] = jax.lax.bitcast_convert_type(d, jnp.int32)
        return carry

    jax.lax.fori_loop(0, NCH // 4, dist_chunk, 0)

    lane32 = jax.lax.broadcasted_iota(jnp.int32, (QT, _KPAD), 1)
    iota0 = jax.lax.broadcasted_iota(jnp.int32, (QT, CH), 1)

    def round_body(r, carry):
        lastidx, idxs, sqs = carry

        def scan_chunk(j2, acc):
            accv, acci = acc
            for u in range(2):
                j = 2 * j2 + u
                gidx = iota0 + j * CH
                kv = jnp.where(gidx == lastidx, IMAX, keys_ref[j])
                keys_ref[j] = kv
                take = kv < accv
                accv = jnp.where(take, kv, accv)
                acci = jnp.where(take, gidx, acci)
            return accv, acci

        acc0 = (jnp.full((QT, CH), IMAX, jnp.int32),
                jnp.full((QT, CH), IMAX, jnp.int32))
        accv, acci = jax.lax.fori_loop(0, NCH // 2, scan_chunk, acc0)
        m = jnp.min(accv, axis=1, keepdims=True)
        am = jnp.min(jnp.where(accv == m, acci, IMAX), axis=1, keepdims=True)
        sel = lane32 == r
        idxs = jnp.where(sel, am, idxs)
        sqs = jnp.where(sel, jax.lax.bitcast_convert_type(m, jnp.float32), sqs)
        return am, idxs, sqs

    init = (jnp.full((QT, 1), -1, jnp.int32),
            jnp.zeros((QT, _KPAD), jnp.int32),
            jnp.zeros((QT, _KPAD), jnp.float32))
    _, idxs, sqs = jax.lax.fori_loop(0, k, round_body, init)
    oi_ref[...] = idxs
    os_ref[...] = sqs


def _knn(q, r, k):
    Nq, Nr = q.shape[0], r.shape[0]
    CH = min(_CH, Nr)
    NCH = Nr // CH
    rt = r.T.reshape(3, NCH, CH).transpose(1, 0, 2)
    grid = (Nq // _QT,)
    oi, os = pl.pallas_call(
        functools.partial(_knn_body, k, NCH, CH),
        grid=grid,
        in_specs=[
            pl.BlockSpec((_QT, 3), lambda i: (i, 0)),
            pl.BlockSpec((NCH, 3, CH), lambda i: (0, 0, 0)),
        ],
        out_specs=[
            pl.BlockSpec((_QT, _KPAD), lambda i: (i, 0)),
            pl.BlockSpec((_QT, _KPAD), lambda i: (i, 0)),
        ],
        out_shape=[
            jax.ShapeDtypeStruct((Nq, _KPAD), jnp.int32),
            jax.ShapeDtypeStruct((Nq, _KPAD), jnp.float32),
        ],
        scratch_shapes=[pltpu.VMEM((NCH, _QT, CH), jnp.int32)],
    )(q, rt)
    return oi[:, :k], os[:, :k]


def _fps_body(n_sample, N, R, coords_ref, poss_ref, out_ref):
    x = coords_ref[0]
    y = coords_ref[1]
    z = coords_ref[2]
    gidx = (jax.lax.broadcasted_iota(jnp.int32, (8, R), 0) * R
            + jax.lax.broadcasted_iota(jnp.int32, (8, R), 1))
    out_ref[0] = 0

    def step(i, dists):
        last = out_ref[i - 1]
        dx = x - poss_ref[0, last]
        dy = y - poss_ref[1, last]
        dz = z - poss_ref[2, last]
        d = (dx * dx + dy * dy) + dz * dz
        dists = jnp.minimum(dists, d)
        m = jnp.max(dists)
        nxt = jnp.min(jnp.where(dists == m, gidx, jnp.int32(N)))
        out_ref[i] = nxt
        return dists

    dists0 = jnp.full((8, R), jnp.inf, jnp.float32)
    jax.lax.fori_loop(1, n_sample, step, dists0)


def _fps(pos, n_sample):
    N = pos.shape[0]
    R = N // 8
    posT = pos.T
    coords = posT.reshape(3, 8, R)
    return pl.pallas_call(
        functools.partial(_fps_body, n_sample, N, R),
        in_specs=[
            pl.BlockSpec(memory_space=pltpu.VMEM),
            pl.BlockSpec(memory_space=pltpu.SMEM),
        ],
        out_specs=pl.BlockSpec(memory_space=pltpu.SMEM),
        out_shape=jax.ShapeDtypeStruct((n_sample,), jnp.int32),
    )(coords, posT)


def _pad128(a):
    D = a.shape[1]
    Dp = -(-D // 128) * 128
    if Dp == D:
        return a
    return jnp.pad(a, ((0, 0), (0, Dp - D)))


def _elu(v):
    poly = v * (1.0 + v * (0.5 + v * (1.0 / 6.0 + v * (1.0 / 24.0
                                                       + v * (1.0 / 120.0)))))
    em1 = jnp.where(v > -0.1, poly, jnp.exp(v) - 1.0)
    return jnp.where(v > 0, v, em1)


def _xconv_body(K, cd, cin, cout, g_ref, pos_ref, r3k_ref, w1t_ref, b1_ref,
                w2t_ref, b2_ref, w2l_ref, linb_ref, c1wt_ref, c1b_ref,
                c2wt_ref, c2b_ref, bh_ref, bx_ref, beff_ref, out_ref,
                tsc, hsc):
    NT = pos_ref.shape[0]
    ptile = pos_ref[...]
    f32 = jnp.float32

    t1 = _elu(jnp.dot(r3k_ref[...], w2l_ref[...],
                      preferred_element_type=f32) + linb_ref[...])
    for g in range(K):
        tg = t1[:, g * K:(g + 1) * K]
        tg = _elu(jnp.dot(tg, c1wt_ref[g],
                          preferred_element_type=f32) + c1b_ref[g])
        tsc[g] = jnp.dot(tg, c2wt_ref[g],
                         preferred_element_type=f32) + c2b_ref[g]

    g3 = g_ref[:, :, 0:3]
    relall = (g3 - jnp.broadcast_to(ptile[None], (K, NT, 3))).reshape(K * NT, 3)
    h = _elu(jnp.dot(relall, w1t_ref[...], preferred_element_type=f32)
             + b1_ref[...])
    h = _elu(jnp.dot(h, w2t_ref[...], preferred_element_type=f32) + b2_ref[...])
    hsc[...] = h.reshape(K, NT, cd)

    KG = 2 if (cd + cin) > 160 else 4
    acc = jnp.zeros((NT, cout), f32)
    for k0 in range(0, K, KG):
        def jb(i, a, k0=k0):
            ahs = list(a[:KG])
            axs = list(a[KG:])
            for u in range(2):
                j = 2 * i + u
                hj = hsc[j]
                xj = g_ref[j, :, 3:3 + cin]
                tj = tsc[j]
                for q in range(KG):
                    w = tj[:, k0 + q:k0 + q + 1]
                    ahs[q] = ahs[q] + hj * w
                    axs[q] = axs[q] + xj * w
            return tuple(ahs) + tuple(axs)

        z = (tuple(jnp.zeros((NT, cd), f32) for _ in range(KG))
             + tuple(jnp.zeros((NT, cin), f32) for _ in range(KG)))
        res = jax.lax.fori_loop(0, K // 2, jb, z)
        for q in range(KG):
            acc = acc + jnp.dot(res[q], bh_ref[k0 + q],
                                preferred_element_type=f32)
            acc = acc + jnp.dot(res[KG + q], bx_ref[k0 + q],
                                preferred_element_type=f32)
    out_ref[...] = jnp.maximum(acc + beff_ref[...], 0.0)


def _xconv(p, x, pos, K):
    N = pos.shape[0]
    cin = x.shape[1]
    cd = p['mlp1_w1'].shape[0]
    cout = p['conv_lin_w'].shape[0]
    C = cin + cd
    dm = -(-cout // C)
    nbr, _ = _knn(pos, pos, K)
    idxf = nbr.T.reshape(-1)
    g = _sc_gather([pos, x], idxf)
    Dp = g.shape[1]
    gk = g.reshape(K, N, Dp)
    pg = _sc_gather([pos], nbr.reshape(-1))
    r3k = (pg[:, :3].reshape(N, K, 3) - pos[:, None, :]).reshape(N, 3 * K)

    w1t = p['mlp1_w1'].T
    w2t = p['mlp1_w2'].T
    w2l = p['mlp2_lin_w'].T
    c1wt = jnp.transpose(p['mlp2_c1_w'], (0, 2, 1))
    c2wt = jnp.transpose(p['mlp2_c2_w'], (0, 2, 1))
    c1b = p['mlp2_c1_b'].reshape(K, 1, K)
    c2b = p['mlp2_c2_b'].reshape(K, 1, K)
    lw = p['conv_lin_w'].reshape(cout, C, dm)
    B = jnp.einsum('cmk,ocm->kco', p['conv_dw_w'], lw)
    bh, bx = B[:, :cd], B[:, cd:]
    beff = (p['conv_lin_b'] + p['conv_dw_b'] @ p['conv_lin_w'].T).reshape(1, cout)

    NT = 128 if C <= 128 else 64
    z = lambda i: (0, 0)
    z3 = lambda i: (0, 0, 0)
    return pl.pallas_call(
        functools.partial(_xconv_body, K, cd, cin, cout),
        grid=(N // NT,),
        in_specs=[
            pl.BlockSpec((K, NT, Dp), lambda i: (0, i, 0)),
            pl.BlockSpec((NT, 3), lambda i: (i, 0)),
            pl.BlockSpec((NT, 3 * K), lambda i: (i, 0)),
            pl.BlockSpec(w1t.shape, z),
            pl.BlockSpec((1, cd), z),
            pl.BlockSpec(w2t.shape, z),
            pl.BlockSpec((1, cd), z),
            pl.BlockSpec(w2l.shape, z),
            pl.BlockSpec((1, K * K), z),
            pl.BlockSpec(c1wt.shape, z3),
            pl.BlockSpec(c1b.shape, z3),
            pl.BlockSpec(c2wt.shape, z3),
            pl.BlockSpec(c2b.shape, z3),
            pl.BlockSpec(bh.shape, z3),
            pl.BlockSpec(bx.shape, z3),
            pl.BlockSpec((1, cout), z),
        ],
        out_specs=pl.BlockSpec((NT, cout), lambda i: (i, 0)),
        out_shape=jax.ShapeDtypeStruct((N, cout), jnp.float32),
        scratch_shapes=[
            pltpu.VMEM((K, NT, K), jnp.float32),
            pltpu.VMEM((K, NT, cd), jnp.float32),
        ],
    )(gk, pos, r3k, w1t, p['mlp1_b1'].reshape(1, cd), w2t,
      p['mlp1_b2'].reshape(1, cd), w2l, p['mlp2_lin_b'].reshape(1, K * K),
      c1wt, c1b, c2wt, c2b, bh, bx, beff)


def _rcp(x):
    r = 1.0 / x
    return r * (2.0 - x * r)


def _interp_body(k, xg_ref, sq_ref, out_ref):
    NT = sq_ref.shape[0]
    Cp = xg_ref.shape[2]
    num = jnp.zeros((NT, Cp), jnp.float32)
    den = jnp.zeros((NT, 1), jnp.float32)
    for j in range(k):
        w = _rcp(jnp.maximum(sq_ref[:, j:j + 1], 1e-16))
        num = num + xg_ref[j] * w
        den = den + w
    out_ref[...] = num * _rcp(den)


def _knn_interpolate(x, pos_x, pos_y, k):
    Ny = pos_y.shape[0]
    idx, sq = _knn(pos_y, pos_x, k)
    sqp = jnp.pad(sq, ((0, 0), (0, _KPAD - k)))
    xg = _sc_gather([x], idx.T.reshape(-1))
    Cp = xg.shape[1]
    xg3 = xg.reshape(k, Ny, Cp)
    NT = 128
    out = pl.pallas_call(
        functools.partial(_interp_body, k),
        grid=(Ny // NT,),
        in_specs=[
            pl.BlockSpec((k, NT, Cp), lambda i: (0, i, 0)),
            pl.BlockSpec((NT, _KPAD), lambda i: (i, 0)),
        ],
        out_specs=pl.BlockSpec((NT, Cp), lambda i: (i, 0)),
        out_shape=jax.ShapeDtypeStruct((Ny, Cp), jnp.float32),
    )(xg3, sqp)
    return out[:, :x.shape[1]]


def _preprocess(x):
    mean3 = jnp.mean(x[:, :3], axis=0)
    xc = jnp.concatenate([x[:, :3] - mean3, x[:, 3:]], axis=1)
    cov = (xc[:, :3].T @ xc[:, :3]) / xc.shape[0]
    _, eigvecs = jnp.linalg.eigh(cov)
    R = eigvecs[:, ::-1]
    xr = jnp.concatenate([xc[:, :3] @ R, xc[:, 3:]], axis=1)
    pos = xr[:, :3]
    return xr, pos


def _final_linear_body(x_ref, w_ref, b_ref, o_ref):
    o_ref[...] = x_ref[...] @ w_ref[...].T + b_ref[...][None, :]


def _final_linear(x, w, b):
    return pl.pallas_call(
        _final_linear_body,
        out_shape=jax.ShapeDtypeStruct((x.shape[0], w.shape[0]), x.dtype),
    )(x, w, b)


def kernel(data_in, params):
    x, pos = _preprocess(data_in)
    pos1 = pos
    x = _xconv(params['enc1'], x, pos, 16)
    idx = _fps(pos, pos.shape[0] // 2)
    g = _sc_gather([pos, x], idx)
    pos, x = g[:, :3], g[:, 3:3 + x.shape[1]]
    pos2 = pos
    x = _xconv(params['enc2'], x, pos, 20)
    idx = _fps(pos, pos.shape[0] // 2)
    g = _sc_gather([pos, x], idx)
    pos, x = g[:, :3], g[:, 3:3 + x.shape[1]]
    x = _xconv(params['enc3'], x, pos, 20)
    x = _xconv(params['enc4'], x, pos, 20)
    x = _xconv(params['dec1'], x, pos, 20)
    x = _knn_interpolate(x, pos, pos2, 16)
    pos = pos2
    x = _xconv(params['dec2'], x, pos, 20)
    x = _knn_interpolate(x, pos, pos1, 16)
    pos = pos1
    x = _xconv(params['dec3'], x, pos, 20)
    return _final_linear(x, params['lin4_w'], params['lin4_b'])


# speedup vs baseline: 1.0283x; 1.0283x over previous
import functools

import jax
import jax.numpy as jnp
from jax import lax
from jax.experimental import pallas as pl
from jax.experimental.pallas import tpu as pltpu
from jax.experimental.pallas import tpu_sc as plsc

_NW = 32
_SC_MESH = dict(core_axis_name="c", subcore_axis_name="s")


def _sc_chunk(bpw, words):
    cb = bpw
    while cb * words * 4 > 320 * 1024:
        cb //= 2
    return cb


def _sc_gather(parts, idx):
    table = _pad128(parts[0] if len(parts) == 1 else jnp.concatenate(parts, axis=1))
    B = idx.shape[0]
    D = table.shape[1]
    bpw = B // _NW
    cb = _sc_chunk(bpw, D)
    nch = bpw // cb
    mesh = plsc.VectorSubcoreMesh(**_SC_MESH)

    @functools.partial(
        pl.kernel, mesh=mesh,
        out_type=jax.ShapeDtypeStruct((B, D), jnp.float32),
        scratch_types=[
            pltpu.VMEM((bpw,), jnp.int32),
            pltpu.VMEM((cb, D), jnp.float32),
            pltpu.SemaphoreType.DMA,
        ],
    )
    def gk(tab, idx_hbm, out, idx_v, rows, sem):
        wid = lax.axis_index("s") * 2 + lax.axis_index("c")
        base = wid * bpw
        pltpu.sync_copy(idx_hbm.at[pl.ds(base, bpw)], idx_v)
        for c in range(nch):
            pltpu.async_copy(tab.at[idx_v.at[pl.ds(c * cb, cb)]], rows, sem).wait()
            pltpu.sync_copy(rows, out.at[pl.ds(base + c * cb, cb)])

    return gk(table, idx)


_QT = 128
_CH = 128
_KPAD = 32


def _knn_body(k, NCH, CH, q_ref, rt_ref, oi_ref, os_ref, keys_ref):
    q = q_ref[...]
    q2 = jnp.sum(q * q, axis=1, keepdims=True)
    QT = q.shape[0]
    IMAX = jnp.int32(0x7FFFFFFF)

    def dist_chunk(j2, carry):
        for u in range(4):
            j = 4 * j2 + u
            rt = rt_ref[j]
            r2 = jnp.sum(rt * rt, axis=0, keepdims=True)
            dot = jnp.dot(q, rt, preferred_element_type=jnp.float32)
            d = jnp.maximum((q2 + r2) - 2.0 * dot, 0.0)
            keys_ref[j] = jax.lax.bitcast_convert_type(d, jnp.int32)
        return carry

    jax.lax.fori_loop(0, NCH // 4, dist_chunk, 0)

    lane32 = jax.lax.broadcasted_iota(jnp.int32, (QT, _KPAD), 1)
    iota0 = jax.lax.broadcasted_iota(jnp.int32, (QT, CH), 1)

    def round_body(r, carry):
        lastidx, idxs, sqs = carry

        def scan_chunk(j2, acc):
            accv, acci = acc
            for u in range(4):
                j = 4 * j2 + u
                gidx = iota0 + j * CH
                kv = jnp.where(gidx == lastidx, IMAX, keys_ref[j])
                keys_ref[j] = kv
                take = kv < accv
                accv = jnp.where(take, kv, accv)
                acci = jnp.where(take, gidx, acci)
            return accv, acci

        acc0 = (jnp.full((QT, CH), IMAX, jnp.int32),
                jnp.full((QT, CH), IMAX, jnp.int32))
        accv, acci = jax.lax.fori_loop(0, NCH // 4, scan_chunk, acc0)
        m = jnp.min(accv, axis=1, keepdims=True)
        am = jnp.min(jnp.where(accv == m, acci, IMAX), axis=1, keepdims=True)
        sel = lane32 == r
        idxs = jnp.where(sel, am, idxs)
        sqs = jnp.where(sel, jax.lax.bitcast_convert_type(m, jnp.float32), sqs)
        return am, idxs, sqs

    init = (jnp.full((QT, 1), -1, jnp.int32),
            jnp.zeros((QT, _KPAD), jnp.int32),
            jnp.zeros((QT, _KPAD), jnp.float32))
    _, idxs, sqs = jax.lax.fori_loop(0, k, round_body, init)
    oi_ref[...] = idxs
    os_ref[...] = sqs


def _knn(q, r, k):
    Nq, Nr = q.shape[0], r.shape[0]
    CH = min(_CH, Nr)
    NCH = Nr // CH
    rt = r.T.reshape(3, NCH, CH).transpose(1, 0, 2)
    grid = (Nq // _QT,)
    oi, os = pl.pallas_call(
        functools.partial(_knn_body, k, NCH, CH),
        grid=grid,
        in_specs=[
            pl.BlockSpec((_QT, 3), lambda i: (i, 0)),
            pl.BlockSpec((NCH, 3, CH), lambda i: (0, 0, 0)),
        ],
        out_specs=[
            pl.BlockSpec((_QT, _KPAD), lambda i: (i, 0)),
            pl.BlockSpec((_QT, _KPAD), lambda i: (i, 0)),
        ],
        out_shape=[
            jax.ShapeDtypeStruct((Nq, _KPAD), jnp.int32),
            jax.ShapeDtypeStruct((Nq, _KPAD), jnp.float32),
        ],
        scratch_shapes=[pltpu.VMEM((NCH, _QT, CH), jnp.int32)],
    )(q, rt)
    return oi[:, :k], os[:, :k]


def _fps_body(n_sample, N, R, coords_ref, poss_ref, out_ref):
    x = coords_ref[0]
    y = coords_ref[1]
    z = coords_ref[2]
    gidx = (jax.lax.broadcasted_iota(jnp.int32, (8, R), 0) * R
            + jax.lax.broadcasted_iota(jnp.int32, (8, R), 1))
    out_ref[0] = 0

    def step(i, dists):
        last = out_ref[i - 1]
        dx = x - poss_ref[0, last]
        dy = y - poss_ref[1, last]
        dz = z - poss_ref[2, last]
        d = (dx * dx + dy * dy) + dz * dz
        dists = jnp.minimum(dists, d)
        m = jnp.max(dists)
        nxt = jnp.min(jnp.where(dists == m, gidx, jnp.int32(N)))
        out_ref[i] = nxt
        return dists

    dists0 = jnp.full((8, R), jnp.inf, jnp.float32)
    jax.lax.fori_loop(1, n_sample, step, dists0)


def _fps(pos, n_sample):
    N = pos.shape[0]
    R = N // 8
    posT = pos.T
    coords = posT.reshape(3, 8, R)
    return pl.pallas_call(
        functools.partial(_fps_body, n_sample, N, R),
        in_specs=[
            pl.BlockSpec(memory_space=pltpu.VMEM),
            pl.BlockSpec(memory_space=pltpu.SMEM),
        ],
        out_specs=pl.BlockSpec(memory_space=pltpu.SMEM),
        out_shape=jax.ShapeDtypeStruct((n_sample,), jnp.int32),
    )(coords, posT)


def _pad128(a):
    D = a.shape[1]
    Dp = -(-D // 128) * 128
    if Dp == D:
        return a
    return jnp.pad(a, ((0, 0), (0, Dp - D)))


def _elu(v):
    poly = v * (1.0 + v * (0.5 + v * (1.0 / 6.0 + v * (1.0 / 24.0
                                                       + v * (1.0 / 120.0)))))
    em1 = jnp.where(v > -0.1, poly, jnp.exp(v) - 1.0)
    return jnp.where(v > 0, v, em1)


def _xconv_body(K, cd, cin, cout, g_ref, pos_ref, r3k_ref, w1t_ref, b1_ref,
                w2t_ref, b2_ref, w2l_ref, linb_ref, c1wt_ref, c1b_ref,
                c2wt_ref, c2b_ref, bh_ref, bx_ref, beff_ref, out_ref,
                tsc, hsc):
    NT = pos_ref.shape[0]
    ptile = pos_ref[...]
    f32 = jnp.float32

    t1 = _elu(jnp.dot(r3k_ref[...], w2l_ref[...],
                      preferred_element_type=f32) + linb_ref[...])
    for g in range(K):
        tg = t1[:, g * K:(g + 1) * K]
        tg = _elu(jnp.dot(tg, c1wt_ref[g],
                          preferred_element_type=f32) + c1b_ref[g])
        tsc[g] = jnp.dot(tg, c2wt_ref[g],
                         preferred_element_type=f32) + c2b_ref[g]

    g3 = g_ref[:, :, 0:3]
    relall = (g3 - jnp.broadcast_to(ptile[None], (K, NT, 3))).reshape(K * NT, 3)
    h = _elu(jnp.dot(relall, w1t_ref[...], preferred_element_type=f32)
             + b1_ref[...])
    h = _elu(jnp.dot(h, w2t_ref[...], preferred_element_type=f32) + b2_ref[...])
    hsc[...] = h.reshape(K, NT, cd)

    KG = 2 if (cd + cin) > 160 else 4
    acc = jnp.zeros((NT, cout), f32)
    for k0 in range(0, K, KG):
        def jb(i, a, k0=k0):
            ahs = list(a[:KG])
            axs = list(a[KG:])
            for u in range(2):
                j = 2 * i + u
                hj = hsc[j]
                xj = g_ref[j, :, 3:3 + cin]
                tj = tsc[j]
                for q in range(KG):
                    w = tj[:, k0 + q:k0 + q + 1]
                    ahs[q] = ahs[q] + hj * w
                    axs[q] = axs[q] + xj * w
            return tuple(ahs) + tuple(axs)

        z = (tuple(jnp.zeros((NT, cd), f32) for _ in range(KG))
             + tuple(jnp.zeros((NT, cin), f32) for _ in range(KG)))
        res = jax.lax.fori_loop(0, K // 2, jb, z)
        for q in range(KG):
            acc = acc + jnp.dot(res[q], bh_ref[k0 + q],
                                preferred_element_type=f32)
            acc = acc + jnp.dot(res[KG + q], bx_ref[k0 + q],
                                preferred_element_type=f32)
    out_ref[...] = jnp.maximum(acc + beff_ref[...], 0.0)


def _xconv(p, x, pos, K):
    N = pos.shape[0]
    cin = x.shape[1]
    cd = p['mlp1_w1'].shape[0]
    cout = p['conv_lin_w'].shape[0]
    C = cin + cd
    dm = -(-cout // C)
    nbr, _ = _knn(pos, pos, K)
    idxf = nbr.T.reshape(-1)
    g = _sc_gather([pos, x], idxf)
    Dp = g.shape[1]
    gk = g.reshape(K, N, Dp)
    pg = _sc_gather([pos], nbr.reshape(-1))
    r3k = (pg[:, :3].reshape(N, K, 3) - pos[:, None, :]).reshape(N, 3 * K)

    w1t = p['mlp1_w1'].T
    w2t = p['mlp1_w2'].T
    w2l = p['mlp2_lin_w'].T
    c1wt = jnp.transpose(p['mlp2_c1_w'], (0, 2, 1))
    c2wt = jnp.transpose(p['mlp2_c2_w'], (0, 2, 1))
    c1b = p['mlp2_c1_b'].reshape(K, 1, K)
    c2b = p['mlp2_c2_b'].reshape(K, 1, K)
    lw = p['conv_lin_w'].reshape(cout, C, dm)
    B = jnp.einsum('cmk,ocm->kco', p['conv_dw_w'], lw)
    bh, bx = B[:, :cd], B[:, cd:]
    beff = (p['conv_lin_b'] + p['conv_dw_b'] @ p['conv_lin_w'].T).reshape(1, cout)

    NT = 128 if C <= 128 else 64
    z = lambda i: (0, 0)
    z3 = lambda i: (0, 0, 0)
    return pl.pallas_call(
        functools.partial(_xconv_body, K, cd, cin, cout),
        grid=(N // NT,),
        in_specs=[
            pl.BlockSpec((K, NT, Dp), lambda i: (0, i, 0)),
            pl.BlockSpec((NT, 3), lambda i: (i, 0)),
            pl.BlockSpec((NT, 3 * K), lambda i: (i, 0)),
            pl.BlockSpec(w1t.shape, z),
            pl.BlockSpec((1, cd), z),
            pl.BlockSpec(w2t.shape, z),
            pl.BlockSpec((1, cd), z),
            pl.BlockSpec(w2l.shape, z),
            pl.BlockSpec((1, K * K), z),
            pl.BlockSpec(c1wt.shape, z3),
            pl.BlockSpec(c1b.shape, z3),
            pl.BlockSpec(c2wt.shape, z3),
            pl.BlockSpec(c2b.shape, z3),
            pl.BlockSpec(bh.shape, z3),
            pl.BlockSpec(bx.shape, z3),
            pl.BlockSpec((1, cout), z),
        ],
        out_specs=pl.BlockSpec((NT, cout), lambda i: (i, 0)),
        out_shape=jax.ShapeDtypeStruct((N, cout), jnp.float32),
        scratch_shapes=[
            pltpu.VMEM((K, NT, K), jnp.float32),
            pltpu.VMEM((K, NT, cd), jnp.float32),
        ],
    )(gk, pos, r3k, w1t, p['mlp1_b1'].reshape(1, cd), w2t,
      p['mlp1_b2'].reshape(1, cd), w2l, p['mlp2_lin_b'].reshape(1, K * K),
      c1wt, c1b, c2wt, c2b, bh, bx, beff)


def _rcp(x):
    r = 1.0 / x
    return r * (2.0 - x * r)


def _interp_body(k, xg_ref, sq_ref, out_ref):
    NT = sq_ref.shape[0]
    Cp = xg_ref.shape[2]
    num = jnp.zeros((NT, Cp), jnp.float32)
    den = jnp.zeros((NT, 1), jnp.float32)
    for j in range(k):
        w = _rcp(jnp.maximum(sq_ref[:, j:j + 1], 1e-16))
        num = num + xg_ref[j] * w
        den = den + w
    out_ref[...] = num * _rcp(den)


def _knn_interpolate(x, pos_x, pos_y, k):
    Ny = pos_y.shape[0]
    idx, sq = _knn(pos_y, pos_x, k)
    sqp = jnp.pad(sq, ((0, 0), (0, _KPAD - k)))
    xg = _sc_gather([x], idx.T.reshape(-1))
    Cp = xg.shape[1]
    xg3 = xg.reshape(k, Ny, Cp)
    NT = 128
    out = pl.pallas_call(
        functools.partial(_interp_body, k),
        grid=(Ny // NT,),
        in_specs=[
            pl.BlockSpec((k, NT, Cp), lambda i: (0, i, 0)),
            pl.BlockSpec((NT, _KPAD), lambda i: (i, 0)),
        ],
        out_specs=pl.BlockSpec((NT, Cp), lambda i: (i, 0)),
        out_shape=jax.ShapeDtypeStruct((Ny, Cp), jnp.float32),
    )(xg3, sqp)
    return out[:, :x.shape[1]]


def _preprocess(x):
    mean3 = jnp.mean(x[:, :3], axis=0)
    xc = jnp.concatenate([x[:, :3] - mean3, x[:, 3:]], axis=1)
    cov = (xc[:, :3].T @ xc[:, :3]) / xc.shape[0]
    _, eigvecs = jnp.linalg.eigh(cov)
    R = eigvecs[:, ::-1]
    xr = jnp.concatenate([xc[:, :3] @ R, xc[:, 3:]], axis=1)
    pos = xr[:, :3]
    return xr, pos


def _final_linear_body(x_ref, w_ref, b_ref, o_ref):
    o_ref[...] = x_ref[...] @ w_ref[...].T + b_ref[...][None, :]


def _final_linear(x, w, b):
    return pl.pallas_call(
        _final_linear_body,
        out_shape=jax.ShapeDtypeStruct((x.shape[0], w.shape[0]), x.dtype),
    )(x, w, b)


def kernel(data_in, params):
    x, pos = _preprocess(data_in)
    pos1 = pos
    x = _xconv(params['enc1'], x, pos, 16)
    idx = _fps(pos, pos.shape[0] // 2)
    g = _sc_gather([pos, x], idx)
    pos, x = g[:, :3], g[:, 3:3 + x.shape[1]]
    pos2 = pos
    x = _xconv(params['enc2'], x, pos, 20)
    idx = _fps(pos, pos.shape[0] // 2)
    g = _sc_gather([pos, x], idx)
    pos, x = g[:, :3], g[:, 3:3 + x.shape[1]]
    x = _xconv(params['enc3'], x, pos, 20)
    x = _xconv(params['enc4'], x, pos, 20)
    x = _xconv(params['dec1'], x, pos, 20)
    x = _knn_interpolate(x, pos, pos2, 16)
    pos = pos2
    x = _xconv(params['dec2'], x, pos, 20)
    x = _knn_interpolate(x, pos, pos1, 16)
    pos = pos1
    x = _xconv(params['dec3'], x, pos, 20)
    return _final_linear(x, params['lin4_w'], params['lin4_b'])


# speedup vs baseline: 1.0446x; 1.0159x over previous
import functools

import jax
import jax.numpy as jnp
from jax import lax
from jax.experimental import pallas as pl
from jax.experimental.pallas import tpu as pltpu
from jax.experimental.pallas import tpu_sc as plsc

_NW = 32
_SC_MESH = dict(core_axis_name="c", subcore_axis_name="s")


def _sc_chunk(bpw, words):
    cb = bpw
    while cb * words * 4 > 320 * 1024:
        cb //= 2
    return cb


def _sc_gather(parts, idx):
    table = _pad128(parts[0] if len(parts) == 1 else jnp.concatenate(parts, axis=1))
    B = idx.shape[0]
    D = table.shape[1]
    bpw = B // _NW
    cb = _sc_chunk(bpw, D)
    nch = bpw // cb
    mesh = plsc.VectorSubcoreMesh(**_SC_MESH)

    @functools.partial(
        pl.kernel, mesh=mesh,
        out_type=jax.ShapeDtypeStruct((B, D), jnp.float32),
        scratch_types=[
            pltpu.VMEM((bpw,), jnp.int32),
            pltpu.VMEM((cb, D), jnp.float32),
            pltpu.SemaphoreType.DMA,
        ],
    )
    def gk(tab, idx_hbm, out, idx_v, rows, sem):
        wid = lax.axis_index("s") * 2 + lax.axis_index("c")
        base = wid * bpw
        pltpu.sync_copy(idx_hbm.at[pl.ds(base, bpw)], idx_v)
        for c in range(nch):
            pltpu.async_copy(tab.at[idx_v.at[pl.ds(c * cb, cb)]], rows, sem).wait()
            pltpu.sync_copy(rows, out.at[pl.ds(base + c * cb, cb)])

    return gk(table, idx)


_QT = 128
_CH = 128
_KPAD = 32


def _knn_body(k, NCH, CH, q_ref, rt_ref, oi_ref, os_ref, keys_ref):
    q = q_ref[...]
    q2 = jnp.sum(q * q, axis=1, keepdims=True)
    QT = q.shape[0]
    IMAX = jnp.int32(0x7FFFFFFF)

    def dist_chunk(j2, carry):
        for u in range(4):
            j = 4 * j2 + u
            rt = rt_ref[j]
            r2 = jnp.sum(rt * rt, axis=0, keepdims=True)
            dot = jnp.dot(q, rt, preferred_element_type=jnp.float32)
            d = jnp.maximum((q2 + r2) - 2.0 * dot, 0.0)
            keys_ref[j] = jax.lax.bitcast_convert_type(d, jnp.int32)
        return carry

    jax.lax.fori_loop(0, NCH // 4, dist_chunk, 0)

    lane32 = jax.lax.broadcasted_iota(jnp.int32, (QT, _KPAD), 1)
    iota0 = jax.lax.broadcasted_iota(jnp.int32, (QT, CH), 1)

    def round_body(r, carry):
        lastidx, idxs, sqs = carry

        def scan_chunk(j2, acc):
            accv, acci = acc
            for u in range(8):
                j = 8 * j2 + u
                gidx = iota0 + j * CH
                kv = jnp.where(gidx == lastidx, IMAX, keys_ref[j])
                keys_ref[j] = kv
                take = kv < accv
                accv = jnp.where(take, kv, accv)
                acci = jnp.where(take, gidx, acci)
            return accv, acci

        acc0 = (jnp.full((QT, CH), IMAX, jnp.int32),
                jnp.full((QT, CH), IMAX, jnp.int32))
        accv, acci = jax.lax.fori_loop(0, NCH // 8, scan_chunk, acc0)
        m = jnp.min(accv, axis=1, keepdims=True)
        am = jnp.min(jnp.where(accv == m, acci, IMAX), axis=1, keepdims=True)
        sel = lane32 == r
        idxs = jnp.where(sel, am, idxs)
        sqs = jnp.where(sel, jax.lax.bitcast_convert_type(m, jnp.float32), sqs)
        return am, idxs, sqs

    init = (jnp.full((QT, 1), -1, jnp.int32),
            jnp.zeros((QT, _KPAD), jnp.int32),
            jnp.zeros((QT, _KPAD), jnp.float32))
    _, idxs, sqs = jax.lax.fori_loop(0, k, round_body, init)
    oi_ref[...] = idxs
    os_ref[...] = sqs


def _knn(q, r, k):
    Nq, Nr = q.shape[0], r.shape[0]
    CH = min(_CH, Nr)
    NCH = Nr // CH
    rt = r.T.reshape(3, NCH, CH).transpose(1, 0, 2)
    grid = (Nq // _QT,)
    oi, os = pl.pallas_call(
        functools.partial(_knn_body, k, NCH, CH),
        grid=grid,
        in_specs=[
            pl.BlockSpec((_QT, 3), lambda i: (i, 0)),
            pl.BlockSpec((NCH, 3, CH), lambda i: (0, 0, 0)),
        ],
        out_specs=[
            pl.BlockSpec((_QT, _KPAD), lambda i: (i, 0)),
            pl.BlockSpec((_QT, _KPAD), lambda i: (i, 0)),
        ],
        out_shape=[
            jax.ShapeDtypeStruct((Nq, _KPAD), jnp.int32),
            jax.ShapeDtypeStruct((Nq, _KPAD), jnp.float32),
        ],
        scratch_shapes=[pltpu.VMEM((NCH, _QT, CH), jnp.int32)],
    )(q, rt)
    return oi[:, :k], os[:, :k]


def _fps_body(n_sample, N, R, coords_ref, poss_ref, out_ref):
    x = coords_ref[0]
    y = coords_ref[1]
    z = coords_ref[2]
    gidx = (jax.lax.broadcasted_iota(jnp.int32, (8, R), 0) * R
            + jax.lax.broadcasted_iota(jnp.int32, (8, R), 1))
    out_ref[0] = 0

    def step(i, dists):
        last = out_ref[i - 1]
        dx = x - poss_ref[0, last]
        dy = y - poss_ref[1, last]
        dz = z - poss_ref[2, last]
        d = (dx * dx + dy * dy) + dz * dz
        dists = jnp.minimum(dists, d)
        m = jnp.max(dists)
        nxt = jnp.min(jnp.where(dists == m, gidx, jnp.int32(N)))
        out_ref[i] = nxt
        return dists

    dists0 = jnp.full((8, R), jnp.inf, jnp.float32)
    jax.lax.fori_loop(1, n_sample, step, dists0)


def _fps(pos, n_sample):
    N = pos.shape[0]
    R = N // 8
    posT = pos.T
    coords = posT.reshape(3, 8, R)
    return pl.pallas_call(
        functools.partial(_fps_body, n_sample, N, R),
        in_specs=[
            pl.BlockSpec(memory_space=pltpu.VMEM),
            pl.BlockSpec(memory_space=pltpu.SMEM),
        ],
        out_specs=pl.BlockSpec(memory_space=pltpu.SMEM),
        out_shape=jax.ShapeDtypeStruct((n_sample,), jnp.int32),
    )(coords, posT)


def _pad128(a):
    D = a.shape[1]
    Dp = -(-D // 128) * 128
    if Dp == D:
        return a
    return jnp.pad(a, ((0, 0), (0, Dp - D)))


def _elu(v):
    poly = v * (1.0 + v * (0.5 + v * (1.0 / 6.0 + v * (1.0 / 24.0
                                                       + v * (1.0 / 120.0)))))
    em1 = jnp.where(v > -0.1, poly, jnp.exp(v) - 1.0)
    return jnp.where(v > 0, v, em1)


def _xconv_body(K, cd, cin, cout, g_ref, pos_ref, r3k_ref, w1t_ref, b1_ref,
                w2t_ref, b2_ref, w2l_ref, linb_ref, c1wt_ref, c1b_ref,
                c2wt_ref, c2b_ref, bh_ref, bx_ref, beff_ref, out_ref,
                tsc, hsc):
    NT = pos_ref.shape[0]
    ptile = pos_ref[...]
    f32 = jnp.float32

    t1 = _elu(jnp.dot(r3k_ref[...], w2l_ref[...],
                      preferred_element_type=f32) + linb_ref[...])
    for g in range(K):
        tg = t1[:, g * K:(g + 1) * K]
        tg = _elu(jnp.dot(tg, c1wt_ref[g],
                          preferred_element_type=f32) + c1b_ref[g])
        tsc[g] = jnp.dot(tg, c2wt_ref[g],
                         preferred_element_type=f32) + c2b_ref[g]

    g3 = g_ref[:, :, 0:3]
    relall = (g3 - jnp.broadcast_to(ptile[None], (K, NT, 3))).reshape(K * NT, 3)
    h = _elu(jnp.dot(relall, w1t_ref[...], preferred_element_type=f32)
             + b1_ref[...])
    h = _elu(jnp.dot(h, w2t_ref[...], preferred_element_type=f32) + b2_ref[...])
    hsc[...] = h.reshape(K, NT, cd)

    KG = 2 if (cd + cin) > 160 else 4
    acc = jnp.zeros((NT, cout), f32)
    for k0 in range(0, K, KG):
        def jb(i, a, k0=k0):
            ahs = list(a[:KG])
            axs = list(a[KG:])
            for u in range(2):
                j = 2 * i + u
                hj = hsc[j]
                xj = g_ref[j, :, 3:3 + cin]
                tj = tsc[j]
                for q in range(KG):
                    w = tj[:, k0 + q:k0 + q + 1]
                    ahs[q] = ahs[q] + hj * w
                    axs[q] = axs[q] + xj * w
            return tuple(ahs) + tuple(axs)

        z = (tuple(jnp.zeros((NT, cd), f32) for _ in range(KG))
             + tuple(jnp.zeros((NT, cin), f32) for _ in range(KG)))
        res = jax.lax.fori_loop(0, K // 2, jb, z)
        for q in range(KG):
            acc = acc + jnp.dot(res[q], bh_ref[k0 + q],
                                preferred_element_type=f32)
            acc = acc + jnp.dot(res[KG + q], bx_ref[k0 + q],
                                preferred_element_type=f32)
    out_ref[...] = jnp.maximum(acc + beff_ref[...], 0.0)


def _xconv(p, x, pos, K):
    N = pos.shape[0]
    cin = x.shape[1]
    cd = p['mlp1_w1'].shape[0]
    cout = p['conv_lin_w'].shape[0]
    C = cin + cd
    dm = -(-cout // C)
    nbr, _ = _knn(pos, pos, K)
    idxf = nbr.T.reshape(-1)
    g = _sc_gather([pos, x], idxf)
    Dp = g.shape[1]
    gk = g.reshape(K, N, Dp)
    pg = _sc_gather([pos], nbr.reshape(-1))
    r3k = (pg[:, :3].reshape(N, K, 3) - pos[:, None, :]).reshape(N, 3 * K)

    w1t = p['mlp1_w1'].T
    w2t = p['mlp1_w2'].T
    w2l = p['mlp2_lin_w'].T
    c1wt = jnp.transpose(p['mlp2_c1_w'], (0, 2, 1))
    c2wt = jnp.transpose(p['mlp2_c2_w'], (0, 2, 1))
    c1b = p['mlp2_c1_b'].reshape(K, 1, K)
    c2b = p['mlp2_c2_b'].reshape(K, 1, K)
    lw = p['conv_lin_w'].reshape(cout, C, dm)
    B = jnp.einsum('cmk,ocm->kco', p['conv_dw_w'], lw)
    bh, bx = B[:, :cd], B[:, cd:]
    beff = (p['conv_lin_b'] + p['conv_dw_b'] @ p['conv_lin_w'].T).reshape(1, cout)

    NT = 128 if C <= 128 else 64
    z = lambda i: (0, 0)
    z3 = lambda i: (0, 0, 0)
    return pl.pallas_call(
        functools.partial(_xconv_body, K, cd, cin, cout),
        grid=(N // NT,),
        in_specs=[
            pl.BlockSpec((K, NT, Dp), lambda i: (0, i, 0)),
            pl.BlockSpec((NT, 3), lambda i: (i, 0)),
            pl.BlockSpec((NT, 3 * K), lambda i: (i, 0)),
            pl.BlockSpec(w1t.shape, z),
            pl.BlockSpec((1, cd), z),
            pl.BlockSpec(w2t.shape, z),
            pl.BlockSpec((1, cd), z),
            pl.BlockSpec(w2l.shape, z),
            pl.BlockSpec((1, K * K), z),
            pl.BlockSpec(c1wt.shape, z3),
            pl.BlockSpec(c1b.shape, z3),
            pl.BlockSpec(c2wt.shape, z3),
            pl.BlockSpec(c2b.shape, z3),
            pl.BlockSpec(bh.shape, z3),
            pl.BlockSpec(bx.shape, z3),
            pl.BlockSpec((1, cout), z),
        ],
        out_specs=pl.BlockSpec((NT, cout), lambda i: (i, 0)),
        out_shape=jax.ShapeDtypeStruct((N, cout), jnp.float32),
        scratch_shapes=[
            pltpu.VMEM((K, NT, K), jnp.float32),
            pltpu.VMEM((K, NT, cd), jnp.float32),
        ],
    )(gk, pos, r3k, w1t, p['mlp1_b1'].reshape(1, cd), w2t,
      p['mlp1_b2'].reshape(1, cd), w2l, p['mlp2_lin_b'].reshape(1, K * K),
      c1wt, c1b, c2wt, c2b, bh, bx, beff)


def _rcp(x):
    r = 1.0 / x
    return r * (2.0 - x * r)


def _interp_body(k, xg_ref, sq_ref, out_ref):
    NT = sq_ref.shape[0]
    Cp = xg_ref.shape[2]
    num = jnp.zeros((NT, Cp), jnp.float32)
    den = jnp.zeros((NT, 1), jnp.float32)
    for j in range(k):
        w = _rcp(jnp.maximum(sq_ref[:, j:j + 1], 1e-16))
        num = num + xg_ref[j] * w
        den = den + w
    out_ref[...] = num * _rcp(den)


def _knn_interpolate(x, pos_x, pos_y, k):
    Ny = pos_y.shape[0]
    idx, sq = _knn(pos_y, pos_x, k)
    sqp = jnp.pad(sq, ((0, 0), (0, _KPAD - k)))
    xg = _sc_gather([x], idx.T.reshape(-1))
    Cp = xg.shape[1]
    xg3 = xg.reshape(k, Ny, Cp)
    NT = 128
    out = pl.pallas_call(
        functools.partial(_interp_body, k),
        grid=(Ny // NT,),
        in_specs=[
            pl.BlockSpec((k, NT, Cp), lambda i: (0, i, 0)),
            pl.BlockSpec((NT, _KPAD), lambda i: (i, 0)),
        ],
        out_specs=pl.BlockSpec((NT, Cp), lambda i: (i, 0)),
        out_shape=jax.ShapeDtypeStruct((Ny, Cp), jnp.float32),
    )(xg3, sqp)
    return out[:, :x.shape[1]]


def _preprocess(x):
    mean3 = jnp.mean(x[:, :3], axis=0)
    xc = jnp.concatenate([x[:, :3] - mean3, x[:, 3:]], axis=1)
    cov = (xc[:, :3].T @ xc[:, :3]) / xc.shape[0]
    _, eigvecs = jnp.linalg.eigh(cov)
    R = eigvecs[:, ::-1]
    xr = jnp.concatenate([xc[:, :3] @ R, xc[:, 3:]], axis=1)
    pos = xr[:, :3]
    return xr, pos


def _final_linear_body(x_ref, w_ref, b_ref, o_ref):
    o_ref[...] = x_ref[...] @ w_ref[...].T + b_ref[...][None, :]


def _final_linear(x, w, b):
    return pl.pallas_call(
        _final_linear_body,
        out_shape=jax.ShapeDtypeStruct((x.shape[0], w.shape[0]), x.dtype),
    )(x, w, b)


def kernel(data_in, params):
    x, pos = _preprocess(data_in)
    pos1 = pos
    x = _xconv(params['enc1'], x, pos, 16)
    idx = _fps(pos, pos.shape[0] // 2)
    g = _sc_gather([pos, x], idx)
    pos, x = g[:, :3], g[:, 3:3 + x.shape[1]]
    pos2 = pos
    x = _xconv(params['enc2'], x, pos, 20)
    idx = _fps(pos, pos.shape[0] // 2)
    g = _sc_gather([pos, x], idx)
    pos, x = g[:, :3], g[:, 3:3 + x.shape[1]]
    x = _xconv(params['enc3'], x, pos, 20)
    x = _xconv(params['enc4'], x, pos, 20)
    x = _xconv(params['dec1'], x, pos, 20)
    x = _knn_interpolate(x, pos, pos2, 16)
    pos = pos2
    x = _xconv(params['dec2'], x, pos, 20)
    x = _knn_interpolate(x, pos, pos1, 16)
    pos = pos1
    x = _xconv(params['dec3'], x, pos, 20)
    return _final_linear(x, params['lin4_w'], params['lin4_b'])
